# Initial kernel scaffold; baseline (speedup 1.0000x reference)
import jax
import jax.numpy as jnp
from jax import lax
from jax.experimental import pallas as pl
from jax.experimental.pallas import tpu as pltpu
from jax.experimental.pallas import tpu_sc as plsc

N = 10000
E = 160000
G = 64
H = 512
EPS = 1e-5
NC = 2
NS = 16
CHUNK = 128
NCHUNK = 4
EB = 200
ROWS_PER_TILE = N // NS
BN = 1000
NBLK = N // BN

_SQRT2 = 1.4142135623730951


def _gelu(x):
    return 0.5 * x * (1.0 + lax.erf(x / _SQRT2))


def _dot(a, b):
    return jax.lax.dot_general(a, b, (((1,), (0,)), ((), ())),
                               preferred_element_type=jnp.float32,
                               precision=jax.lax.Precision.HIGHEST)


def _dis_from_deg(deg_blk):
    deg = deg_blk[0, :, 0] + deg_blk[1, :, 0] + 1.0
    return lax.rsqrt(deg)


def _onehot(batch_blk):
    g = lax.broadcasted_iota(jnp.int32, (batch_blk.shape[0], G), 1)
    return (batch_blk[:, None] == g).astype(jnp.float32)


def _ln_stats(stats):
    cnt = jnp.maximum(stats[2, :], 1.0)
    norm = cnt * float(H)
    mean = stats[0, :] / norm
    var = stats[1, :] / norm - mean * mean
    rstd = lax.rsqrt(var + EPS)
    return cnt, mean, rstd


def _sc_mesh():
    return plsc.VectorSubcoreMesh(core_axis_name="c", subcore_axis_name="s")


def _deg_body(dst_hbm, zeros_hbm, ones_hbm, out_hbm, idx_v, upd_v, acc_sh):
    c = lax.axis_index("c")
    s = lax.axis_index("s")
    wid = c * NS + s
    base = wid * (E // (NC * NS))
    rbase = s * ROWS_PER_TILE
    pltpu.sync_copy(zeros_hbm.at[pl.ds(rbase, ROWS_PER_TILE)],
                    acc_sh.at[pl.ds(rbase, ROWS_PER_TILE)])
    pltpu.sync_copy(ones_hbm, upd_v)
    plsc.subcore_barrier()

    def body(i, _):
        off = pl.multiple_of(base + i * EB, 8)
        pltpu.sync_copy(dst_hbm.at[pl.ds(off, EB)], idx_v)
        pltpu.sync_copy(upd_v, acc_sh.at[idx_v], add=True)
        return 0

    lax.fori_loop(0, E // (NC * NS) // EB, body, 0)
    plsc.subcore_barrier()
    pltpu.sync_copy(acc_sh.at[pl.ds(rbase, ROWS_PER_TILE)],
                    out_hbm.at[c, pl.ds(rbase, ROWS_PER_TILE)])


def _sc_degree(dst, zeros_n16, ones_b16):
    k = pl.kernel(
        _deg_body,
        out_type=jax.ShapeDtypeStruct((NC, N, 16), jnp.float32),
        mesh=_sc_mesh(),
        scratch_types=[
            pltpu.VMEM((EB,), jnp.int32),
            pltpu.VMEM((EB, 16), jnp.float32),
            pltpu.VMEM_SHARED((N, 16), jnp.float32),
        ],
    )
    return k(dst, zeros_n16, ones_b16)


def _agg_body(y0, y1, y2, y3, src_hbm, dst_hbm, a0, a1, a2, a3,
              sidx_v, didx_v, rows_v, sem, acc_sh):
    c = lax.axis_index("c")
    s = lax.axis_index("s")
    base = s * (E // NS)
    rbase = s * ROWS_PER_TILE
    ys = [y0, y1, y2, y3]
    outs = [a0, a1, a2, a3]

    for chunk in range(NCHUNK):
        mine = (chunk % NC) == c

        @pl.when(mine)
        def _(chunk=chunk):
            pltpu.sync_copy(ys[chunk].at[pl.ds(rbase, ROWS_PER_TILE)],
                            acc_sh.at[pl.ds(rbase, ROWS_PER_TILE)])
        plsc.subcore_barrier()

        @pl.when(mine)
        def _(chunk=chunk):
            y = ys[chunk]

            def body(i, _):
                off = pl.multiple_of(base + i * EB, 8)
                pltpu.sync_copy(src_hbm.at[pl.ds(off, EB)], sidx_v)
                pltpu.sync_copy(dst_hbm.at[pl.ds(off, EB)], didx_v)
                pltpu.async_copy(y.at[sidx_v], rows_v, sem).wait()
                pltpu.sync_copy(rows_v, acc_sh.at[didx_v], add=True)
                return 0

            lax.fori_loop(0, E // NS // EB, body, 0)
        plsc.subcore_barrier()

        @pl.when(mine)
        def _(chunk=chunk):
            pltpu.sync_copy(acc_sh.at[pl.ds(rbase, ROWS_PER_TILE)],
                            outs[chunk].at[pl.ds(rbase, ROWS_PER_TILE)])
        plsc.subcore_barrier()


def _sc_aggregate(y_chunks, src, dst):
    out_t = [jax.ShapeDtypeStruct((N, CHUNK), jnp.float32)
             for _ in range(NCHUNK)]
    k = pl.kernel(
        _agg_body,
        out_type=out_t,
        mesh=_sc_mesh(),
        scratch_types=[
            pltpu.VMEM((EB,), jnp.int32),
            pltpu.VMEM((EB,), jnp.int32),
            pltpu.VMEM((EB, CHUNK), jnp.float32),
            pltpu.SemaphoreType.DMA,
            pltpu.VMEM_SHARED((N, CHUNK), jnp.float32),
        ],
    )
    return k(*y_chunks, src, dst)


def _a1_body(x_ref, deg_ref, w_ref, y0, y1, y2, y3):
    dis = _dis_from_deg(deg_ref[...])
    y = _dot(x_ref[...], w_ref[...]) * dis[:, None]
    y0[...] = y[:, 0*CHUNK:1*CHUNK]
    y1[...] = y[:, 1*CHUNK:2*CHUNK]
    y2[...] = y[:, 2*CHUNK:3*CHUNK]
    y3[...] = y[:, 3*CHUNK:4*CHUNK]


def _tc_first_matmul(x, deg2, W1):
    inx = x.shape[1]
    yspec = pl.BlockSpec((BN, CHUNK), lambda i: (i, 0))
    return pl.pallas_call(
        _a1_body,
        grid=(NBLK,),
        in_specs=[
            pl.BlockSpec((BN, inx), lambda i: (i, 0)),
            pl.BlockSpec((NC, BN, 16), lambda i: (0, i, 0)),
            pl.BlockSpec((inx, H), lambda i: (0, 0)),
        ],
        out_specs=[yspec, yspec, yspec, yspec],
        out_shape=[jax.ShapeDtypeStruct((N, CHUNK), jnp.float32)] * 4,
    )(x, deg2, W1)


def _c_body(a0, a1, a2, a3, deg_ref, b_ref, batch_ref, h_ref, st_ref):
    i = pl.program_id(0)
    dis = _dis_from_deg(deg_ref[...])
    agg = jnp.concatenate([a0[...], a1[...], a2[...], a3[...]], axis=1)
    out = agg * dis[:, None] + b_ref[...]
    h_ref[...] = out
    rs = jnp.sum(out, axis=1)
    rq = jnp.sum(out * out, axis=1)
    ones = jnp.ones((BN,), jnp.float32)
    zeros = jnp.zeros((BN,), jnp.float32)
    m = jnp.stack([rs, rq, ones, zeros, zeros, zeros, zeros, zeros], axis=0)
    contrib = _dot(m, _onehot(batch_ref[0, 0, :]))

    @pl.when(i == 0)
    def _():
        st_ref[...] = jnp.zeros_like(st_ref)

    st_ref[...] += contrib


def _tc_post_agg(a_chunks, deg2, b_row, batch3d):
    aspec = pl.BlockSpec((BN, CHUNK), lambda i: (i, 0))
    return pl.pallas_call(
        _c_body,
        grid=(NBLK,),
        in_specs=[
            aspec, aspec, aspec, aspec,
            pl.BlockSpec((NC, BN, 16), lambda i: (0, i, 0)),
            pl.BlockSpec((1, H), lambda i: (0, 0)),
            pl.BlockSpec((1, 1, BN), lambda i: (i, 0, 0)),
        ],
        out_specs=[
            pl.BlockSpec((BN, H), lambda i: (i, 0)),
            pl.BlockSpec((8, G), lambda i: (0, 0)),
        ],
        out_shape=[
            jax.ShapeDtypeStruct((N, H), jnp.float32),
            jax.ShapeDtypeStruct((8, G), jnp.float32),
        ],
    )(*a_chunks, deg2, b_row, batch3d)


def _a23_body(h_ref, st_ref, batch_ref, deg_ref, g_ref, bt_ref, w_ref,
              y0, y1, y2, y3):
    _, mean, rstd = _ln_stats(st_ref[...])
    oh = _onehot(batch_ref[0, 0, :])
    mr = _dot(oh, jnp.stack([mean, rstd], axis=1))
    xn = (h_ref[...] - mr[:, 0:1]) * mr[:, 1:2] * g_ref[...] + bt_ref[...]
    hact = _gelu(xn)
    dis = _dis_from_deg(deg_ref[...])
    y = _dot(hact, w_ref[...]) * dis[:, None]
    y0[...] = y[:, 0*CHUNK:1*CHUNK]
    y1[...] = y[:, 1*CHUNK:2*CHUNK]
    y2[...] = y[:, 2*CHUNK:3*CHUNK]
    y3[...] = y[:, 3*CHUNK:4*CHUNK]


def _tc_mid_matmul(h, st, batch3d, deg2, gamma_row, beta_row, W):
    yspec = pl.BlockSpec((BN, CHUNK), lambda i: (i, 0))
    return pl.pallas_call(
        _a23_body,
        grid=(NBLK,),
        in_specs=[
            pl.BlockSpec((BN, H), lambda i: (i, 0)),
            pl.BlockSpec((8, G), lambda i: (0, 0)),
            pl.BlockSpec((1, 1, BN), lambda i: (i, 0, 0)),
            pl.BlockSpec((NC, BN, 16), lambda i: (0, i, 0)),
            pl.BlockSpec((1, H), lambda i: (0, 0)),
            pl.BlockSpec((1, H), lambda i: (0, 0)),
            pl.BlockSpec((H, H), lambda i: (0, 0)),
        ],
        out_specs=[yspec, yspec, yspec, yspec],
        out_shape=[jax.ShapeDtypeStruct((N, CHUNK), jnp.float32)] * 4,
    )(h, st, batch3d, deg2, gamma_row, beta_row, W)


def _e_body(h_ref, st_ref, batch_ref, g_ref, bt_ref,
            wr_ref, br_ref, wh_ref, bh_ref, wo_ref, bo_ref,
            out_ref, acc_ref):
    i = pl.program_id(0)
    cnt, mean, rstd = _ln_stats(st_ref[...])
    oh = _onehot(batch_ref[0, 0, :])
    mr = _dot(oh, jnp.stack([mean, rstd], axis=1))
    xn = (h_ref[...] - mr[:, 0:1]) * mr[:, 1:2] * g_ref[...] + bt_ref[...]
    hact = _gelu(xn)
    pooled = jax.lax.dot_general(
        oh, hact, (((0,), (0,)), ((), ())),
        preferred_element_type=jnp.float32,
        precision=jax.lax.Precision.HIGHEST)

    @pl.when(i == 0)
    def _():
        acc_ref[...] = jnp.zeros_like(acc_ref)

    acc_ref[...] += pooled

    @pl.when(i == NBLK - 1)
    def _():
        g = acc_ref[...] / cnt[:, None]
        g = _gelu(_dot(g, wr_ref[...]) + br_ref[...])
        g = _gelu(_dot(g, wh_ref[...]) + bh_ref[...])
        out_ref[...] = _dot(g, wo_ref[...]) + bo_ref[...]


def _tc_head(h, st, batch3d, gamma_row, beta_row, Wr, br, Wh, bh, Wo, bo):
    out_dim = Wo.shape[1]
    full = lambda a, b: pl.BlockSpec((a, b), lambda i: (0, 0))
    return pl.pallas_call(
        _e_body,
        grid=(NBLK,),
        in_specs=[
            pl.BlockSpec((BN, H), lambda i: (i, 0)),
            pl.BlockSpec((8, G), lambda i: (0, 0)),
            pl.BlockSpec((1, 1, BN), lambda i: (i, 0, 0)),
            full(1, H), full(1, H),
            full(H, H // 2), full(1, H // 2),
            full(H // 2, H // 4), full(1, H // 4),
            full(H // 4, out_dim), full(1, out_dim),
        ],
        out_specs=pl.BlockSpec((G, out_dim), lambda i: (0, 0)),
        out_shape=jax.ShapeDtypeStruct((G, out_dim), jnp.float32),
        scratch_shapes=[pltpu.VMEM((G, H), jnp.float32)],
    )(h, st, batch3d, gamma_row, beta_row, Wr, br, Wh, bh, Wo, bo)


def kernel(x, edge_index, batch, W1, b1, ln1_w, ln1_b, W2, b2, ln2_w, ln2_b,
           W3, b3, ln3_w, ln3_b, Wr, br, Wh, bh, Wo, bo):
    src = edge_index[0]
    dst = edge_index[1]
    batch3d = batch.reshape(NBLK, 1, BN)
    zeros_n16 = jnp.zeros((N, 16), jnp.float32)
    ones_b16 = jnp.ones((EB, 16), jnp.float32)

    deg2 = _sc_degree(dst, zeros_n16, ones_b16)

    row = lambda v: v.reshape(1, -1)

    y = _tc_first_matmul(x, deg2, W1)
    a = _sc_aggregate(y, src, dst)
    h, st = _tc_post_agg(a, deg2, row(b1), batch3d)

    y = _tc_mid_matmul(h, st, batch3d, deg2, row(ln1_w), row(ln1_b), W2)
    a = _sc_aggregate(y, src, dst)
    h, st = _tc_post_agg(a, deg2, row(b2), batch3d)

    y = _tc_mid_matmul(h, st, batch3d, deg2, row(ln2_w), row(ln2_b), W3)
    a = _sc_aggregate(y, src, dst)
    h, st = _tc_post_agg(a, deg2, row(b3), batch3d)

    return _tc_head(h, st, batch3d, row(ln3_w), row(ln3_b),
                    Wr, row(br), Wh, row(bh), Wo, row(bo))


# speedup vs baseline: 6.0757x; 6.0757x over previous
import jax
import jax.numpy as jnp
from jax import lax
from jax.experimental import pallas as pl
from jax.experimental.pallas import tpu as pltpu
from jax.experimental.pallas import tpu_sc as plsc

N = 10000
NPAD = 10240
E = 160000
G = 64
H = 512
EPS = 1e-5
NC = 2
NS = 16
CHUNK = 128
NCHUNK = 4
EB = 80
EBD = 40
ROWS_PER_TILE = NPAD // NS
BN = 1000
NBLK = N // BN

_SQRT2 = 1.4142135623730951


def _gelu(x):
    return 0.5 * x * (1.0 + lax.erf(x / _SQRT2))


def _dot(a, b):
    return jax.lax.dot_general(a, b, (((1,), (0,)), ((), ())),
                               preferred_element_type=jnp.float32,
                               precision=jax.lax.Precision.HIGHEST)


def _dis_from_deg(deg_blk):
    deg = deg_blk[0, :, 0] + deg_blk[1, :, 0] + 1.0
    return lax.rsqrt(deg)


def _onehot(batch_blk):
    g = lax.broadcasted_iota(jnp.int32, (batch_blk.shape[0], G), 1)
    return (batch_blk[:, None] == g).astype(jnp.float32)


def _ln_stats(stats):
    cnt = jnp.maximum(stats[2, :], 1.0)
    norm = cnt * float(H)
    mean = stats[0, :] / norm
    var = stats[1, :] / norm - mean * mean
    rstd = lax.rsqrt(var + EPS)
    return cnt, mean, rstd


def _sc_mesh():
    return plsc.VectorSubcoreMesh(core_axis_name="c", subcore_axis_name="s")


def _deg_body(dst_hbm, zeros_hbm, ones_hbm, out_hbm, idx_v, upd_v, acc_sh):
    c = lax.axis_index("c")
    s = lax.axis_index("s")
    wid = c * NS + s
    base = wid * (E // (NC * NS))
    rbase = s * ROWS_PER_TILE
    pltpu.sync_copy(zeros_hbm.at[pl.ds(rbase, ROWS_PER_TILE)],
                    acc_sh.at[pl.ds(rbase, ROWS_PER_TILE)])
    pltpu.sync_copy(ones_hbm, upd_v)
    plsc.subcore_barrier()

    def body(i, _):
        off = pl.multiple_of(base + i * EBD, 8)
        pltpu.sync_copy(dst_hbm.at[pl.ds(off, EBD)], idx_v)
        pltpu.sync_copy(upd_v, acc_sh.at[idx_v], add=True)
        return 0

    lax.fori_loop(0, E // (NC * NS) // EBD, body, 0)
    plsc.subcore_barrier()
    pltpu.sync_copy(acc_sh.at[pl.ds(rbase, ROWS_PER_TILE)],
                    out_hbm.at[c, pl.ds(rbase, ROWS_PER_TILE)])


def _sc_degree(dst, zeros_n16, ones_b16):
    k = pl.kernel(
        _deg_body,
        out_type=jax.ShapeDtypeStruct((NC, NPAD, CHUNK), jnp.float32),
        mesh=_sc_mesh(),
        scratch_types=[
            pltpu.VMEM((EBD,), jnp.int32),
            pltpu.VMEM((EBD, CHUNK), jnp.float32),
            pltpu.VMEM_SHARED((NPAD, CHUNK), jnp.float32),
        ],
    )
    return k(dst, zeros_n16, ones_b16)


def _agg_body(y0, y1, y2, y3, src_hbm, dst_hbm, a0, a1, a2, a3,
              sidx_v, didx_v, rows_v, sem, acc_sh):
    c = lax.axis_index("c")
    s = lax.axis_index("s")
    base = s * (E // NS)
    rbase = s * ROWS_PER_TILE
    ys = [y0, y1, y2, y3]
    outs = [a0, a1, a2, a3]

    for chunk in range(NCHUNK):
        mine = (chunk % NC) == c

        @pl.when(mine)
        def _(chunk=chunk):
            pltpu.sync_copy(ys[chunk].at[pl.ds(rbase, ROWS_PER_TILE)],
                            acc_sh.at[pl.ds(rbase, ROWS_PER_TILE)])
        plsc.subcore_barrier()

        @pl.when(mine)
        def _(chunk=chunk):
            y = ys[chunk]

            def body(i, _):
                off = pl.multiple_of(base + i * EB, 8)
                pltpu.sync_copy(src_hbm.at[pl.ds(off, EB)], sidx_v)
                pltpu.sync_copy(dst_hbm.at[pl.ds(off, EB)], didx_v)
                pltpu.async_copy(y.at[sidx_v], rows_v, sem).wait()
                pltpu.sync_copy(rows_v, acc_sh.at[didx_v], add=True)
                return 0

            lax.fori_loop(0, E // NS // EB, body, 0)
        plsc.subcore_barrier()

        @pl.when(mine)
        def _(chunk=chunk):
            pltpu.sync_copy(acc_sh.at[pl.ds(rbase, ROWS_PER_TILE)],
                            outs[chunk].at[pl.ds(rbase, ROWS_PER_TILE)])
        plsc.subcore_barrier()


def _sc_aggregate(y_chunks, src, dst):
    out_t = [jax.ShapeDtypeStruct((NPAD, CHUNK), jnp.float32)
             for _ in range(NCHUNK)]
    k = pl.kernel(
        _agg_body,
        out_type=out_t,
        mesh=_sc_mesh(),
        scratch_types=[
            pltpu.VMEM((EB,), jnp.int32),
            pltpu.VMEM((EB,), jnp.int32),
            pltpu.VMEM((EB, CHUNK), jnp.float32),
            pltpu.SemaphoreType.DMA,
            pltpu.VMEM_SHARED((NPAD, CHUNK), jnp.float32),
        ],
    )
    return k(*y_chunks, src, dst)


def _a1_body(x_ref, deg_ref, w_ref, y0, y1, y2, y3):
    dis = _dis_from_deg(deg_ref[...])
    y = _dot(x_ref[...], w_ref[...]) * dis[:, None]
    y0[...] = y[:, 0*CHUNK:1*CHUNK]
    y1[...] = y[:, 1*CHUNK:2*CHUNK]
    y2[...] = y[:, 2*CHUNK:3*CHUNK]
    y3[...] = y[:, 3*CHUNK:4*CHUNK]


def _tc_first_matmul(x, deg2, W1):
    inx = x.shape[1]
    yspec = pl.BlockSpec((BN, CHUNK), lambda i: (i, 0))
    return pl.pallas_call(
        _a1_body,
        grid=(NBLK,),
        in_specs=[
            pl.BlockSpec((BN, inx), lambda i: (i, 0)),
            pl.BlockSpec((NC, BN, CHUNK), lambda i: (0, i, 0)),
            pl.BlockSpec((inx, H), lambda i: (0, 0)),
        ],
        out_specs=[yspec, yspec, yspec, yspec],
        out_shape=[jax.ShapeDtypeStruct((NPAD, CHUNK), jnp.float32)] * 4,
    )(x, deg2, W1)


def _c_body(a0, a1, a2, a3, deg_ref, b_ref, batch_ref, h_ref, st_ref):
    i = pl.program_id(0)
    dis = _dis_from_deg(deg_ref[...])
    agg = jnp.concatenate([a0[...], a1[...], a2[...], a3[...]], axis=1)
    out = agg * dis[:, None] + b_ref[...]
    h_ref[...] = out
    rs = jnp.sum(out, axis=1)
    rq = jnp.sum(out * out, axis=1)
    ones = jnp.ones((BN,), jnp.float32)
    zeros = jnp.zeros((BN,), jnp.float32)
    m = jnp.stack([rs, rq, ones, zeros, zeros, zeros, zeros, zeros], axis=0)
    contrib = _dot(m, _onehot(batch_ref[0, 0, :]))

    @pl.when(i == 0)
    def _():
        st_ref[...] = jnp.zeros_like(st_ref)

    st_ref[...] += contrib


def _tc_post_agg(a_chunks, deg2, b_row, batch3d):
    aspec = pl.BlockSpec((BN, CHUNK), lambda i: (i, 0))
    return pl.pallas_call(
        _c_body,
        grid=(NBLK,),
        in_specs=[
            aspec, aspec, aspec, aspec,
            pl.BlockSpec((NC, BN, CHUNK), lambda i: (0, i, 0)),
            pl.BlockSpec((1, H), lambda i: (0, 0)),
            pl.BlockSpec((1, 1, BN), lambda i: (i, 0, 0)),
        ],
        out_specs=[
            pl.BlockSpec((BN, H), lambda i: (i, 0)),
            pl.BlockSpec((8, G), lambda i: (0, 0)),
        ],
        out_shape=[
            jax.ShapeDtypeStruct((N, H), jnp.float32),
            jax.ShapeDtypeStruct((8, G), jnp.float32),
        ],
    )(*a_chunks, deg2, b_row, batch3d)


def _a23_body(h_ref, st_ref, batch_ref, deg_ref, g_ref, bt_ref, w_ref,
              y0, y1, y2, y3):
    _, mean, rstd = _ln_stats(st_ref[...])
    oh = _onehot(batch_ref[0, 0, :])
    mr = _dot(oh, jnp.stack([mean, rstd], axis=1))
    xn = (h_ref[...] - mr[:, 0:1]) * mr[:, 1:2] * g_ref[...] + bt_ref[...]
    hact = _gelu(xn)
    dis = _dis_from_deg(deg_ref[...])
    y = _dot(hact, w_ref[...]) * dis[:, None]
    y0[...] = y[:, 0*CHUNK:1*CHUNK]
    y1[...] = y[:, 1*CHUNK:2*CHUNK]
    y2[...] = y[:, 2*CHUNK:3*CHUNK]
    y3[...] = y[:, 3*CHUNK:4*CHUNK]


def _tc_mid_matmul(h, st, batch3d, deg2, gamma_row, beta_row, W):
    yspec = pl.BlockSpec((BN, CHUNK), lambda i: (i, 0))
    return pl.pallas_call(
        _a23_body,
        grid=(NBLK,),
        in_specs=[
            pl.BlockSpec((BN, H), lambda i: (i, 0)),
            pl.BlockSpec((8, G), lambda i: (0, 0)),
            pl.BlockSpec((1, 1, BN), lambda i: (i, 0, 0)),
            pl.BlockSpec((NC, BN, CHUNK), lambda i: (0, i, 0)),
            pl.BlockSpec((1, H), lambda i: (0, 0)),
            pl.BlockSpec((1, H), lambda i: (0, 0)),
            pl.BlockSpec((H, H), lambda i: (0, 0)),
        ],
        out_specs=[yspec, yspec, yspec, yspec],
        out_shape=[jax.ShapeDtypeStruct((NPAD, CHUNK), jnp.float32)] * 4,
    )(h, st, batch3d, deg2, gamma_row, beta_row, W)


def _e_body(h_ref, st_ref, batch_ref, g_ref, bt_ref,
            wr_ref, br_ref, wh_ref, bh_ref, wo_ref, bo_ref,
            out_ref, acc_ref):
    i = pl.program_id(0)
    cnt, mean, rstd = _ln_stats(st_ref[...])
    oh = _onehot(batch_ref[0, 0, :])
    mr = _dot(oh, jnp.stack([mean, rstd], axis=1))
    xn = (h_ref[...] - mr[:, 0:1]) * mr[:, 1:2] * g_ref[...] + bt_ref[...]
    hact = _gelu(xn)
    pooled = jax.lax.dot_general(
        oh, hact, (((0,), (0,)), ((), ())),
        preferred_element_type=jnp.float32,
        precision=jax.lax.Precision.HIGHEST)

    @pl.when(i == 0)
    def _():
        acc_ref[...] = jnp.zeros_like(acc_ref)

    acc_ref[...] += pooled

    @pl.when(i == NBLK - 1)
    def _():
        g = acc_ref[...] / cnt[:, None]
        g = _gelu(_dot(g, wr_ref[...]) + br_ref[...])
        g = _gelu(_dot(g, wh_ref[...]) + bh_ref[...])
        out_ref[...] = _dot(g, wo_ref[...]) + bo_ref[...]


def _tc_head(h, st, batch3d, gamma_row, beta_row, Wr, br, Wh, bh, Wo, bo):
    out_dim = Wo.shape[1]
    full = lambda a, b: pl.BlockSpec((a, b), lambda i: (0, 0))
    return pl.pallas_call(
        _e_body,
        grid=(NBLK,),
        in_specs=[
            pl.BlockSpec((BN, H), lambda i: (i, 0)),
            pl.BlockSpec((8, G), lambda i: (0, 0)),
            pl.BlockSpec((1, 1, BN), lambda i: (i, 0, 0)),
            full(1, H), full(1, H),
            full(H, H // 2), full(1, H // 2),
            full(H // 2, H // 4), full(1, H // 4),
            full(H // 4, out_dim), full(1, out_dim),
        ],
        out_specs=pl.BlockSpec((G, out_dim), lambda i: (0, 0)),
        out_shape=jax.ShapeDtypeStruct((G, out_dim), jnp.float32),
        scratch_shapes=[pltpu.VMEM((G, H), jnp.float32)],
    )(h, st, batch3d, gamma_row, beta_row, Wr, br, Wh, bh, Wo, bo)


def kernel(x, edge_index, batch, W1, b1, ln1_w, ln1_b, W2, b2, ln2_w, ln2_b,
           W3, b3, ln3_w, ln3_b, Wr, br, Wh, bh, Wo, bo):
    src = edge_index[0]
    dst = edge_index[1]
    batch3d = batch.reshape(NBLK, 1, BN)
    zeros_n16 = jnp.zeros((NPAD, CHUNK), jnp.float32)
    ones_b16 = jnp.ones((EBD, CHUNK), jnp.float32)

    deg2 = _sc_degree(dst, zeros_n16, ones_b16)

    row = lambda v: v.reshape(1, -1)

    y = _tc_first_matmul(x, deg2, W1)
    a = _sc_aggregate(y, src, dst)
    h, st = _tc_post_agg(a, deg2, row(b1), batch3d)

    y = _tc_mid_matmul(h, st, batch3d, deg2, row(ln1_w), row(ln1_b), W2)
    a = _sc_aggregate(y, src, dst)
    h, st = _tc_post_agg(a, deg2, row(b2), batch3d)

    y = _tc_mid_matmul(h, st, batch3d, deg2, row(ln2_w), row(ln2_b), W3)
    a = _sc_aggregate(y, src, dst)
    h, st = _tc_post_agg(a, deg2, row(b3), batch3d)

    return _tc_head(h, st, batch3d, row(ln3_w), row(ln3_b),
                    Wr, row(br), Wh, row(bh), Wo, row(bo))
